# Initial kernel scaffold; baseline (speedup 1.0000x reference)
import jax
import jax.numpy as jnp
from jax.experimental import pallas as pl


def kernel(X0, k1, k1n, k2, k3, k3n, k4, TA0, TI0, Cinhib0, masks, k5, k5n, k6, kdI, kdT):
    raise NotImplementedError("write your pallas kernel here")


# speedup vs baseline: 6.1617x; 6.1617x over previous
import jax
import jax.numpy as jnp
from jax.experimental import pallas as pl
from jax.experimental.pallas import tpu as pltpu

_L = 4
_B = 1024
_D = 1024
_EPS = 1e-6
_E0 = 1.0
_TILE = 256
_T = _D // _TILE


def _body(x0, k1, k1n, k2, k3, k3n, k4, ta0, ti0, cin0, masks,
          k5, k5n, k6, kdi, kdt, out_ref,
          x_buf, activ, inhib, v_ref):
    l = pl.program_id(0)
    t = pl.program_id(1)

    @pl.when(jnp.logical_and(l == 0, t == 0))
    def _init():
        out_ref[:] = jnp.ones_like(out_ref)
        x_buf[:] = x0[:]

    @pl.when(t < _T)
    def _tile():
        m = masks[0]
        kact = jnp.where(m > 0, ta0[0] * k1[0] / (k1n[0] + k2[0] + _EPS), 0.0)
        kinh = jnp.where(m < 0, ti0[0] * k3[0] / (k3n[0] + k4[0] + _EPS), 0.0)
        cact = k2[0] * kact
        cinh = cin0[0] * k4[0] * kinh
        colsum = jnp.sum(kact + kinh, axis=0, keepdims=True)

        @pl.when(t == 0)
        def _():
            v_ref[:] = colsum

        @pl.when(t > 0)
        def _():
            v_ref[:] = v_ref[:] + colsum

        x = x_buf[:]
        dn = (((1,), (1,)), ((), ()))
        a = jax.lax.dot_general(x, cact, dn, preferred_element_type=jnp.float32)
        b = jax.lax.dot_general(x, cinh, dn, preferred_element_type=jnp.float32)
        activ[:, pl.ds(t * _TILE, _TILE)] = a
        inhib[:, pl.ds(t * _TILE, _TILE)] = b

    @pl.when(t == _T)
    def _finalize():
        x = x_buf[:]
        cp = out_ref[:] + jnp.sum(x * v_ref[:], axis=1, keepdims=True)
        kdtcp = kdt[0] * cp
        kdicp = kdi[0] * cp
        x_eq = _E0 * activ[:] / (kdtcp + _E0 * inhib[:] / kdicp + _EPS)
        w5 = k5[0] / (k5n[0] + k6[0] + _EPS)
        out_ref[:] = cp + jnp.sum(x_eq * w5, axis=1, keepdims=True)
        x_buf[:] = x_eq


def kernel(X0, k1, k1n, k2, k3, k3n, k4, TA0, TI0, Cinhib0, masks,
           k5, k5n, k6, kdI, kdT):
    big = pl.BlockSpec((1, _TILE, _D),
                       lambda l, t: (l, jnp.minimum(t, _T - 1), 0))
    vec = pl.BlockSpec((1, 1, _D), lambda l, t: (l, 0, 0))
    k5, k5n, k6, kdI, kdT = (a.reshape(_L, 1, _D)
                             for a in (k5, k5n, k6, kdI, kdT))
    cp = pl.pallas_call(
        _body,
        grid=(_L, _T + 1),
        in_specs=[pl.BlockSpec((_B, _D), lambda l, t: (0, 0))]
        + [big] * 10 + [vec] * 5,
        out_specs=pl.BlockSpec((_B, 1), lambda l, t: (0, 0)),
        out_shape=jax.ShapeDtypeStruct((_B, 1), jnp.float32),
        scratch_shapes=[
            pltpu.VMEM((_B, _D), jnp.float32),
            pltpu.VMEM((_B, _D), jnp.float32),
            pltpu.VMEM((_B, _D), jnp.float32),
            pltpu.VMEM((1, _D), jnp.float32),
        ],
    )(X0, k1, k1n, k2, k3, k3n, k4, TA0, TI0, Cinhib0, masks,
      k5, k5n, k6, kdI, kdT)
    return cp.reshape(_B)
